# Initial kernel scaffold; baseline (speedup 1.0000x reference)
import jax
import jax.numpy as jnp
from jax.experimental import pallas as pl


def kernel(x, Wg, W1, b1, W2, b2):
    raise NotImplementedError("write your pallas kernel here")


# speedup vs baseline: 3.2530x; 3.2530x over previous
import functools

import jax
import jax.numpy as jnp
from jax import lax
from jax.experimental import pallas as pl
from jax.experimental.pallas import tpu as pltpu
from jax.experimental.pallas import tpu_sc as plsc

NE = 8
T = 2048
H = 768
ID = 3072
TT = 256
IB = 512
NI = ID // IB
NW = 32
RW = T // NW


def _router_body(x_ref, wg_ref, probs_ref, rank_ref, sid_ref, ws_ref, offs_ref):
    x = x_ref[...]
    wg = wg_ref[...]
    logits = lax.dot_general(x, wg, (((1,), (1,)), ((), ())),
                             preferred_element_type=jnp.float32)
    m = jnp.max(logits, axis=-1, keepdims=True)
    ex = jnp.exp(logits - m)
    probs = ex / jnp.sum(ex, axis=-1, keepdims=True)
    probs_ref[...] = probs

    maxp = jnp.max(probs, axis=-1, keepdims=True)
    w = maxp / (maxp + 1e-9)
    eids = lax.broadcasted_iota(jnp.int32, (T, NE), 1)
    idx = jnp.min(jnp.where(probs == maxp, eids, NE), axis=-1, keepdims=True)
    onehot = (eids == idx).astype(jnp.float32)

    ti = lax.broadcasted_iota(jnp.int32, (T, T), 0)
    tj = lax.broadcasted_iota(jnp.int32, (T, T), 1)
    ltri = (ti > tj).astype(jnp.bfloat16)
    pos = lax.dot_general(ltri, onehot.astype(jnp.bfloat16),
                          (((1,), (0,)), ((), ())),
                          preferred_element_type=jnp.float32)
    counts = jnp.sum(onehot, axis=0, keepdims=True)
    counts16 = jnp.concatenate(
        [counts, jnp.zeros((1, 16 - NE), jnp.float32)], axis=1)
    li = lax.broadcasted_iota(jnp.int32, (16, 16), 0)
    lj = lax.broadcasted_iota(jnp.int32, (16, 16), 1)
    upper = (li < lj).astype(jnp.float32)
    starts16 = lax.dot_general(counts16, upper, (((1,), (0,)), ((), ())),
                               preferred_element_type=jnp.float32)
    offs_ref[...] = starts16.astype(jnp.int32)
    rankf = jnp.sum(onehot * (starts16[:, :NE] + pos), axis=-1, keepdims=True)
    rank_ref[...] = rankf.astype(jnp.int32)

    onesrow = jnp.full((1, T), 1.0, jnp.float32)
    diag_rank = (ti == tj).astype(jnp.float32) * rankf
    rankrow = lax.dot_general(onesrow, diag_rank, (((1,), (0,)), ((), ())),
                              preferred_element_type=jnp.float32)
    perm = (rankrow == ti.astype(jnp.float32)).astype(jnp.float32)
    tcol = lax.broadcasted_iota(jnp.int32, (T, 1), 0).astype(jnp.float32)
    sid = lax.dot_general(perm, tcol, (((1,), (0,)), ((), ())),
                          preferred_element_type=jnp.float32)
    sid_ref[...] = sid.astype(jnp.int32)
    ws_ref[...] = lax.dot_general(perm, w, (((1,), (0,)), ((), ())),
                                  preferred_element_type=jnp.float32)


def _router(x_flat, Wg):
    return pl.pallas_call(
        _router_body,
        out_shape=[
            jax.ShapeDtypeStruct((T, NE), jnp.float32),
            jax.ShapeDtypeStruct((T, 1), jnp.int32),
            jax.ShapeDtypeStruct((T, 1), jnp.int32),
            jax.ShapeDtypeStruct((T, 1), jnp.float32),
            jax.ShapeDtypeStruct((1, 16), jnp.int32),
        ],
    )(x_flat, Wg)


@functools.cache
def _make_dispatch():
    mesh = plsc.VectorSubcoreMesh(core_axis_name="c", subcore_axis_name="s")

    @functools.partial(
        pl.kernel,
        mesh=mesh,
        out_type=jax.ShapeDtypeStruct((T, H), jnp.float32),
        scratch_types=[
            pltpu.VMEM((RW,), jnp.int32),
            pltpu.VMEM((RW, H), jnp.float32),
            pltpu.SemaphoreType.DMA,
        ],
    )
    def dispatch(x_hbm, sid_hbm, xs_hbm, sid_l, rows_v, sem):
        wid = lax.axis_index("s") * 2 + lax.axis_index("c")
        base = wid * RW
        pltpu.sync_copy(sid_hbm.at[pl.ds(base, RW)], sid_l)
        pltpu.async_copy(x_hbm.at[sid_l], rows_v, sem).wait()
        pltpu.sync_copy(rows_v, xs_hbm.at[pl.ds(base, RW)])

    return dispatch


def _ffn_body(offs_ref, ws_ref, xs_ref, w1_ref, b1_ref, w2_ref, b2_ref, out_ref):
    e = pl.program_id(0)
    i = pl.program_id(1)

    @pl.when((e == 0) & (i == 0))
    def _():
        out_ref[...] = jnp.zeros_like(out_ref)

    start = offs_ref[0, e]
    stop = offs_ref[0, e + 1]
    t0 = start // TT
    t1 = (stop + TT - 1) // TT
    w1 = w1_ref[0]
    b1 = b1_ref[0]
    w2 = w2_ref[0]
    b2 = b2_ref[0] * (i == 0).astype(jnp.float32)

    def tile(t, c):
        r0 = t * TT
        xt = xs_ref[pl.ds(r0, TT), :]
        h = jnp.dot(xt, w1, preferred_element_type=jnp.float32) + b1
        h = 0.5 * h * (1.0 + lax.erf(h * 0.7071067811865476))
        contrib = jnp.dot(h, w2, preferred_element_type=jnp.float32) + b2
        rows = r0 + lax.broadcasted_iota(jnp.int32, (TT, 1), 0)
        mask = ((rows >= start) & (rows < stop)).astype(jnp.float32)
        wcol = ws_ref[pl.ds(r0, TT), :]
        out_ref[pl.ds(r0, TT), :] += mask * wcol * contrib
        return c
    lax.fori_loop(t0, t1, tile, 0)


def _ffn(offs, ws2, xs, W1, b1, W2, b2):
    return pl.pallas_call(
        _ffn_body,
        grid=(NE, NI),
        in_specs=[
            pl.BlockSpec((1, 16), lambda e, i: (0, 0), memory_space=pltpu.SMEM),
            pl.BlockSpec((T, 1), lambda e, i: (0, 0)),
            pl.BlockSpec((T, H), lambda e, i: (0, 0)),
            pl.BlockSpec((1, H, IB), lambda e, i: (e, 0, i)),
            pl.BlockSpec((1, 1, IB), lambda e, i: (e, 0, i)),
            pl.BlockSpec((1, IB, H), lambda e, i: (e, i, 0)),
            pl.BlockSpec((1, 1, H), lambda e, i: (e, 0, 0)),
        ],
        out_specs=pl.BlockSpec((T, H), lambda e, i: (0, 0)),
        out_shape=jax.ShapeDtypeStruct((T, H), jnp.float32),
        compiler_params=pltpu.CompilerParams(
            dimension_semantics=("arbitrary", "arbitrary")),
    )(offs, ws2, xs, W1, b1.reshape(NE, 1, ID), W2, b2.reshape(NE, 1, H))


@functools.cache
def _make_combine():
    mesh = plsc.VectorSubcoreMesh(core_axis_name="c", subcore_axis_name="s")

    @functools.partial(
        pl.kernel,
        mesh=mesh,
        out_type=jax.ShapeDtypeStruct((T, H), jnp.float32),
        scratch_types=[
            pltpu.VMEM((RW,), jnp.int32),
            pltpu.VMEM((RW, H), jnp.float32),
            pltpu.SemaphoreType.DMA,
        ],
    )
    def combine(ys_hbm, rank_hbm, out_hbm, rk_l, rows_v, sem):
        wid = lax.axis_index("s") * 2 + lax.axis_index("c")
        base = wid * RW
        pltpu.sync_copy(rank_hbm.at[pl.ds(base, RW)], rk_l)
        pltpu.async_copy(ys_hbm.at[rk_l], rows_v, sem).wait()
        pltpu.sync_copy(rows_v, out_hbm.at[pl.ds(base, RW)])

    return combine


def kernel(x, Wg, W1, b1, W2, b2):
    Bb, Ss, Hh = x.shape
    x_flat = x.reshape(T, H)
    probs, rank2, sid2, ws, offs = _router(x_flat, Wg)
    rank = rank2.reshape(T)
    sid = sid2.reshape(T)
    xs = _make_dispatch()(x_flat, sid)
    ys = _ffn(offs, ws, xs, W1, b1, W2, b2)
    out = _make_combine()(ys, rank)
    return out.reshape(Bb, Ss, Hh), probs.reshape(Bb, Ss, NE)
